# initial kernel scaffold (unmeasured)
import functools

import jax
import jax.numpy as jnp
from jax import lax
from jax.experimental import pallas as pl
from jax.experimental.pallas import tpu as pltpu

N_DEV = 4
SQ = 1024
SKV = 1024
D_MODEL = 1024
H_PER = 8
DH = 128
SCALE = 0.08838834764831843
NEG = -1e9


def _body(x_ref, wq_ref, k_ref, v_ref, wo_ref, out_ref,
          comm_ref, send_sems, recv_sems):
    my = lax.axis_index("i")
    left = lax.rem(my + N_DEV - 1, N_DEV)
    right = lax.rem(my + 1, N_DEV)

    barrier_sem = pltpu.get_barrier_semaphore()
    for nbr in (left, right):
        pl.semaphore_signal(
            barrier_sem, inc=1,
            device_id=(nbr,), device_id_type=pl.DeviceIdType.MESH,
        )
    pl.semaphore_wait(barrier_sem, 2)

    comm_ref[my, 0] = wq_ref[...]
    comm_ref[my, 1] = wo_ref[...]

    for h in range(N_DEV - 1):
        src = lax.rem(my - h + N_DEV, N_DEV)
        rdma = pltpu.make_async_remote_copy(
            src_ref=comm_ref.at[src],
            dst_ref=comm_ref.at[src],
            send_sem=send_sems.at[h],
            recv_sem=recv_sems.at[h],
            device_id=(right,),
            device_id_type=pl.DeviceIdType.MESH,
        )
        rdma.start()
        rdma.wait()

    q_i = lax.broadcasted_iota(jnp.int32, (SQ, SKV), 0)
    k_i = lax.broadcasted_iota(jnp.int32, (SQ, SKV), 1)
    mask = ((q_i // 64) % 4) == ((k_i // 64) % 4)

    x = x_ref[0]
    acc = jnp.zeros((SQ, D_MODEL), jnp.float32)
    for g in range(N_DEV):
        q_g = jnp.dot(x, comm_ref[g, 0], preferred_element_type=jnp.float32)
        ctx_cols = []
        for h in range(H_PER):
            head = g * H_PER + h
            q_h = q_g[:, h * DH:(h + 1) * DH]
            k_h = k_ref[0, :, head, :]
            s = jnp.dot(q_h, k_h.T, preferred_element_type=jnp.float32) * SCALE
            s = jnp.where(mask, s, NEG)
            m = jnp.max(s, axis=-1, keepdims=True)
            w = jnp.exp(s - m)
            w = w / jnp.sum(w, axis=-1, keepdims=True)
            v_h = v_ref[0, :, head, :]
            ctx_cols.append(jnp.dot(w, v_h, preferred_element_type=jnp.float32))
        ctx = jnp.concatenate(ctx_cols, axis=1)
        acc = acc + jnp.dot(ctx, comm_ref[g, 1], preferred_element_type=jnp.float32)
    out_ref[0] = acc


def kernel(x, Wq, K_ext, V_ext, Wo):
    return pl.pallas_call(
        _body,
        out_shape=jax.ShapeDtypeStruct((1, SQ, D_MODEL), jnp.float32),
        in_specs=[
            pl.BlockSpec(memory_space=pltpu.VMEM),
            pl.BlockSpec(memory_space=pltpu.VMEM),
            pl.BlockSpec(memory_space=pltpu.VMEM),
            pl.BlockSpec(memory_space=pltpu.VMEM),
            pl.BlockSpec(memory_space=pltpu.VMEM),
        ],
        out_specs=pl.BlockSpec(memory_space=pltpu.VMEM),
        scratch_shapes=[
            pltpu.VMEM((N_DEV, 2, D_MODEL, D_MODEL), jnp.float32),
            pltpu.SemaphoreType.DMA((N_DEV - 1,)),
            pltpu.SemaphoreType.DMA((N_DEV - 1,)),
        ],
        compiler_params=pltpu.CompilerParams(collective_id=0),
    )(x, Wq, K_ext, V_ext, Wo)


# baseline (device time: 442768 ns/iter reference)
import jax
import jax.numpy as jnp
from jax import lax
from jax.experimental import pallas as pl
from jax.experimental.pallas import tpu as pltpu

N_DEV = 4
SQ = 1024
SKV = 1024
D_MODEL = 1024
H_PER = 8
N_HEADS = 32
DH = 128
CHUNK = 256
SCALE = 0.08838834764831843
NEG = -1e9


def _body(x_ref, wq_ref, k_ref, v_ref, wo_ref, out_ref,
          comm_ref, kbuf, vbuf, send_sems, recv_sems, copy_sems, kv_sems):
    my = lax.axis_index("i")
    left = lax.rem(my + N_DEV - 1, N_DEV)
    right = lax.rem(my + 1, N_DEV)

    barrier_sem = pltpu.get_barrier_semaphore()
    for nbr in (left, right):
        pl.semaphore_signal(
            barrier_sem, inc=1,
            device_id=(nbr,), device_id_type=pl.DeviceIdType.MESH,
        )
    pl.semaphore_wait(barrier_sem, 2)

    cq = pltpu.make_async_copy(wq_ref, comm_ref.at[my, 0], copy_sems.at[0])
    co = pltpu.make_async_copy(wo_ref, comm_ref.at[my, 1], copy_sems.at[1])
    cq.start()
    co.start()
    cq.wait()
    co.wait()

    for h in range(N_DEV - 1):
        src = lax.rem(my - h + N_DEV, N_DEV)
        rdma = pltpu.make_async_remote_copy(
            src_ref=comm_ref.at[src],
            dst_ref=comm_ref.at[src],
            send_sem=send_sems.at[h],
            recv_sem=recv_sems.at[h],
            device_id=(right,),
            device_id_type=pl.DeviceIdType.MESH,
        )
        rdma.start()
        rdma.wait()

    q_i = lax.broadcasted_iota(jnp.int32, (CHUNK, SKV), 0)
    k_i = lax.broadcasted_iota(jnp.int32, (CHUNK, SKV), 1)
    mask = ((q_i // 64) % 4) == ((k_i // 64) % 4)

    def kv_dma(t, slot):
        g, h = divmod(t, H_PER)
        head = g * H_PER + h
        ck = pltpu.make_async_copy(
            k_ref.at[0, :, head, :], kbuf.at[slot], kv_sems.at[slot, 0])
        cv = pltpu.make_async_copy(
            v_ref.at[0, :, head, :], vbuf.at[slot], kv_sems.at[slot, 1])
        ck.start()
        cv.start()
        return ck, cv

    x = x_ref[0]
    pending = kv_dma(0, 0)
    q_g = None
    for t in range(N_HEADS):
        g, h = divmod(t, H_PER)
        slot = t % 2
        if h == 0:
            q_g = jnp.dot(x, comm_ref[g, 0], preferred_element_type=jnp.float32)
        pending[0].wait()
        pending[1].wait()
        if t + 1 < N_HEADS:
            pending = kv_dma(t + 1, (t + 1) % 2)
        k_h = kbuf[slot]
        v_h = vbuf[slot]
        wo_h = comm_ref[g, 1, h * DH:(h + 1) * DH, :]
        for c in range(SQ // CHUNK):
            q_hc = q_g[c * CHUNK:(c + 1) * CHUNK, h * DH:(h + 1) * DH]
            s = jnp.dot(q_hc, k_h.T, preferred_element_type=jnp.float32) * SCALE
            s = jnp.where(mask, s, NEG)
            m = jnp.max(s, axis=-1, keepdims=True)
            w = jnp.exp(s - m)
            w = w / jnp.sum(w, axis=-1, keepdims=True)
            ctx = jnp.dot(w, v_h, preferred_element_type=jnp.float32)
            part = jnp.dot(ctx, wo_h, preferred_element_type=jnp.float32)
            if g == 0 and h == 0:
                out_ref[0, c * CHUNK:(c + 1) * CHUNK, :] = part
            else:
                out_ref[0, c * CHUNK:(c + 1) * CHUNK, :] += part


def kernel(x, Wq, K_ext, V_ext, Wo):
    return pl.pallas_call(
        _body,
        out_shape=jax.ShapeDtypeStruct((1, SQ, D_MODEL), jnp.float32),
        in_specs=[
            pl.BlockSpec(memory_space=pltpu.VMEM),
            pl.BlockSpec(memory_space=pl.ANY),
            pl.BlockSpec(memory_space=pl.ANY),
            pl.BlockSpec(memory_space=pl.ANY),
            pl.BlockSpec(memory_space=pl.ANY),
        ],
        out_specs=pl.BlockSpec(memory_space=pltpu.VMEM),
        scratch_shapes=[
            pltpu.VMEM((N_DEV, 2, D_MODEL, D_MODEL), jnp.float32),
            pltpu.VMEM((2, SKV, DH), jnp.float32),
            pltpu.VMEM((2, SKV, DH), jnp.float32),
            pltpu.SemaphoreType.DMA((N_DEV - 1,)),
            pltpu.SemaphoreType.DMA((N_DEV - 1,)),
            pltpu.SemaphoreType.DMA((2,)),
            pltpu.SemaphoreType.DMA((2, 2)),
        ],
        compiler_params=pltpu.CompilerParams(
            collective_id=0,
            vmem_limit_bytes=63 * 1024 * 1024,
        ),
    )(x, Wq, K_ext, V_ext, Wo)


# device time: 191772 ns/iter; 2.3088x vs baseline; 2.3088x over previous
import jax
import jax.numpy as jnp
from jax import lax
from jax.experimental import pallas as pl
from jax.experimental.pallas import tpu as pltpu

N_DEV = 4
SQ = 1024
SKV = 1024
D_MODEL = 1024
H_PER = 8
N_HEADS = 32
DH = 128
CHUNK = 256
SCALE = 0.08838834764831843
NEG = -1e9


def _body(x_ref, wq_ref, k_ref, v_ref, wo_ref, out_ref,
          comm_ref, kbuf, vbuf, send_sems, recv_sems, kv_sems):
    my = lax.axis_index("i")
    left = lax.rem(my + N_DEV - 1, N_DEV)
    right = lax.rem(my + 1, N_DEV)

    comm_ref[0, 0] = wq_ref[...]
    comm_ref[0, 1] = wo_ref[...]

    barrier_sem = pltpu.get_barrier_semaphore()
    for nbr in (left, right):
        pl.semaphore_signal(
            barrier_sem, inc=1,
            device_id=(nbr,), device_id_type=pl.DeviceIdType.MESH,
        )
    pl.semaphore_wait(barrier_sem, 2)

    q_i = lax.broadcasted_iota(jnp.int32, (CHUNK, SKV), 0)
    k_i = lax.broadcasted_iota(jnp.int32, (CHUNK, SKV), 1)
    mask = ((q_i // 64) % 4) == ((k_i // 64) % 4)

    def kv_dma(t, slot):
        s, h = divmod(t, H_PER)
        head = lax.rem(my - s + N_DEV, N_DEV) * H_PER + h
        ck = pltpu.make_async_copy(
            k_ref.at[0, :, head, :], kbuf.at[slot], kv_sems.at[slot, 0])
        cv = pltpu.make_async_copy(
            v_ref.at[0, :, head, :], vbuf.at[slot], kv_sems.at[slot, 1])
        ck.start()
        cv.start()
        return ck, cv

    x = x_ref[0]
    pending = kv_dma(0, 0)
    hops = []
    for s in range(N_DEV):
        if s < N_DEV - 1:
            rdma = pltpu.make_async_remote_copy(
                src_ref=comm_ref.at[s],
                dst_ref=comm_ref.at[s + 1],
                send_sem=send_sems.at[s],
                recv_sem=recv_sems.at[s],
                device_id=(right,),
                device_id_type=pl.DeviceIdType.MESH,
            )
            rdma.start()
            hops.append(rdma)

        q_g = jnp.dot(x, comm_ref[s, 0], preferred_element_type=jnp.float32)
        q_g = q_g.astype(jnp.bfloat16)
        for h in range(H_PER):
            t = s * H_PER + h
            slot = t % 2
            pending[0].wait()
            pending[1].wait()
            if t + 1 < N_HEADS:
                pending = kv_dma(t + 1, (t + 1) % 2)
            k_h = kbuf[slot].astype(jnp.bfloat16)
            v_h = vbuf[slot].astype(jnp.bfloat16)
            wo_h = comm_ref[s, 1, h * DH:(h + 1) * DH, :]
            for c in range(SQ // CHUNK):
                q_hc = q_g[c * CHUNK:(c + 1) * CHUNK, h * DH:(h + 1) * DH]
                sc = jnp.dot(q_hc, k_h.T, preferred_element_type=jnp.float32)
                sc = jnp.where(mask, sc * SCALE, NEG)
                m = jnp.max(sc, axis=-1, keepdims=True)
                w = jnp.exp(sc - m)
                w = (w / jnp.sum(w, axis=-1, keepdims=True)).astype(jnp.bfloat16)
                ctx = jnp.dot(w, v_h, preferred_element_type=jnp.float32)
                part = jnp.dot(ctx.astype(jnp.bfloat16), wo_h,
                               preferred_element_type=jnp.float32)
                if s == 0 and h == 0:
                    out_ref[0, c * CHUNK:(c + 1) * CHUNK, :] = part
                else:
                    out_ref[0, c * CHUNK:(c + 1) * CHUNK, :] += part

        if s < N_DEV - 1:
            hops[s].wait_recv()

    for rdma in hops:
        rdma.wait_send()


def kernel(x, Wq, K_ext, V_ext, Wo):
    x16 = x.astype(jnp.bfloat16)
    wq16 = Wq.astype(jnp.bfloat16)
    wo16 = Wo.astype(jnp.bfloat16)
    return pl.pallas_call(
        _body,
        out_shape=jax.ShapeDtypeStruct((1, SQ, D_MODEL), jnp.float32),
        in_specs=[
            pl.BlockSpec(memory_space=pltpu.VMEM),
            pl.BlockSpec(memory_space=pltpu.VMEM),
            pl.BlockSpec(memory_space=pl.ANY),
            pl.BlockSpec(memory_space=pl.ANY),
            pl.BlockSpec(memory_space=pltpu.VMEM),
        ],
        out_specs=pl.BlockSpec(memory_space=pltpu.VMEM),
        scratch_shapes=[
            pltpu.VMEM((N_DEV, 2, D_MODEL, D_MODEL), jnp.bfloat16),
            pltpu.VMEM((2, SKV, DH), jnp.float32),
            pltpu.VMEM((2, SKV, DH), jnp.float32),
            pltpu.SemaphoreType.DMA((N_DEV - 1,)),
            pltpu.SemaphoreType.DMA((N_DEV - 1,)),
            pltpu.SemaphoreType.DMA((2, 2)),
        ],
        compiler_params=pltpu.CompilerParams(
            collective_id=0,
            vmem_limit_bytes=63 * 1024 * 1024,
        ),
    )(x16, wq16, K_ext, V_ext, wo16)


# device time: 146861 ns/iter; 3.0149x vs baseline; 1.3058x over previous
import jax
import jax.numpy as jnp
from jax import lax
from jax.experimental import pallas as pl
from jax.experimental.pallas import tpu as pltpu

N_DEV = 4
SQ = 1024
SKV = 1024
D_MODEL = 1024
H_PER = 8
H_HALF = 4
N_HEADS = 32
DH = 128
HALF = 512
CHUNK = 256
SCALE = 0.08838834764831843
NEG = -1e9


def _body(x_ref, wq_ref, k_ref, v_ref, wo_ref, out_ref,
          wqa, woa, wqb, wob, kbuf, vbuf, send_sems, recv_sems, kv_sems):
    my = lax.axis_index("i")
    left = lax.rem(my + N_DEV - 1, N_DEV)
    right = lax.rem(my + 1, N_DEV)

    wqa[0] = wq_ref[:, :HALF]
    wqb[0] = wq_ref[:, HALF:]
    woa[0] = wo_ref[:HALF, :]
    wob[0] = wo_ref[HALF:, :]

    barrier_sem = pltpu.get_barrier_semaphore()
    for nbr in (left, right):
        pl.semaphore_signal(
            barrier_sem, inc=1,
            device_id=(nbr,), device_id_type=pl.DeviceIdType.MESH,
        )
    pl.semaphore_wait(barrier_sem, 2)

    q_i = lax.broadcasted_iota(jnp.int32, (CHUNK, SKV), 0)
    k_i = lax.broadcasted_iota(jnp.int32, (CHUNK, SKV), 1)
    mask = ((q_i // 64) % 4) == ((k_i // 64) % 4)

    def head_index(s, h):
        if h < H_HALF:
            return lax.rem(my - s + N_DEV, N_DEV) * H_PER + h
        return lax.rem(my + s, N_DEV) * H_PER + h

    def kv_dma(t, slot):
        s, h = divmod(t, H_PER)
        head = head_index(s, h)
        ck = pltpu.make_async_copy(
            k_ref.at[0, :, head, :], kbuf.at[slot], kv_sems.at[slot, 0])
        cv = pltpu.make_async_copy(
            v_ref.at[0, :, head, :], vbuf.at[slot], kv_sems.at[slot, 1])
        ck.start()
        cv.start()
        return ck, cv

    x = x_ref[0]
    pending = kv_dma(0, 0)
    hops = []
    for s in range(N_DEV):
        if s < N_DEV - 1:
            stage_hops = []
            for j, (buf, dev) in enumerate(
                    [(wqa, right), (woa, right), (wqb, left), (wob, left)]):
                rdma = pltpu.make_async_remote_copy(
                    src_ref=buf.at[s],
                    dst_ref=buf.at[s + 1],
                    send_sem=send_sems.at[s, j],
                    recv_sem=recv_sems.at[s, j],
                    device_id=(dev,),
                    device_id_type=pl.DeviceIdType.MESH,
                )
                rdma.start()
                stage_hops.append(rdma)
            hops.append(stage_hops)

        q_a = jnp.dot(x, wqa[s], preferred_element_type=jnp.float32)
        q_a = q_a.astype(jnp.bfloat16)
        q_b = jnp.dot(x, wqb[s], preferred_element_type=jnp.float32)
        q_b = q_b.astype(jnp.bfloat16)
        for h in range(H_PER):
            t = s * H_PER + h
            slot = t % 2
            pending[0].wait()
            pending[1].wait()
            if t + 1 < N_HEADS:
                pending = kv_dma(t + 1, (t + 1) % 2)
            k_h = kbuf[slot].astype(jnp.bfloat16)
            v_h = vbuf[slot].astype(jnp.bfloat16)
            if h < H_HALF:
                q_h = q_a[:, h * DH:(h + 1) * DH]
                wo_h = woa[s, h * DH:(h + 1) * DH, :]
            else:
                q_h = q_b[:, (h - H_HALF) * DH:(h - H_HALF + 1) * DH]
                wo_h = wob[s, (h - H_HALF) * DH:(h - H_HALF + 1) * DH, :]
            for c in range(SQ // CHUNK):
                q_hc = q_h[c * CHUNK:(c + 1) * CHUNK, :]
                sc = jnp.dot(q_hc, k_h.T, preferred_element_type=jnp.float32)
                sc = jnp.where(mask, sc * SCALE, NEG)
                m = jnp.max(sc, axis=-1, keepdims=True)
                w = jnp.exp(sc - m)
                w = (w / jnp.sum(w, axis=-1, keepdims=True)).astype(jnp.bfloat16)
                ctx = jnp.dot(w, v_h, preferred_element_type=jnp.float32)
                part = jnp.dot(ctx.astype(jnp.bfloat16), wo_h,
                               preferred_element_type=jnp.float32)
                if s == 0 and h == 0:
                    out_ref[0, c * CHUNK:(c + 1) * CHUNK, :] = part
                else:
                    out_ref[0, c * CHUNK:(c + 1) * CHUNK, :] += part

        if s < N_DEV - 1:
            for rdma in hops[s]:
                rdma.wait_recv()

    for stage_hops in hops:
        for rdma in stage_hops:
            rdma.wait_send()


def kernel(x, Wq, K_ext, V_ext, Wo):
    x16 = x.astype(jnp.bfloat16)
    wq16 = Wq.astype(jnp.bfloat16)
    wo16 = Wo.astype(jnp.bfloat16)
    return pl.pallas_call(
        _body,
        out_shape=jax.ShapeDtypeStruct((1, SQ, D_MODEL), jnp.float32),
        in_specs=[
            pl.BlockSpec(memory_space=pltpu.VMEM),
            pl.BlockSpec(memory_space=pltpu.VMEM),
            pl.BlockSpec(memory_space=pl.ANY),
            pl.BlockSpec(memory_space=pl.ANY),
            pl.BlockSpec(memory_space=pltpu.VMEM),
        ],
        out_specs=pl.BlockSpec(memory_space=pltpu.VMEM),
        scratch_shapes=[
            pltpu.VMEM((N_DEV, D_MODEL, HALF), jnp.bfloat16),
            pltpu.VMEM((N_DEV, HALF, D_MODEL), jnp.bfloat16),
            pltpu.VMEM((N_DEV, D_MODEL, HALF), jnp.bfloat16),
            pltpu.VMEM((N_DEV, HALF, D_MODEL), jnp.bfloat16),
            pltpu.VMEM((2, SKV, DH), jnp.float32),
            pltpu.VMEM((2, SKV, DH), jnp.float32),
            pltpu.SemaphoreType.DMA((N_DEV - 1, 4)),
            pltpu.SemaphoreType.DMA((N_DEV - 1, 4)),
            pltpu.SemaphoreType.DMA((2, 2)),
        ],
        compiler_params=pltpu.CompilerParams(
            collective_id=0,
            vmem_limit_bytes=63 * 1024 * 1024,
        ),
    )(x16, wq16, K_ext, V_ext, wo16)


# device time: 128258 ns/iter; 3.4522x vs baseline; 1.1450x over previous
import jax
import jax.numpy as jnp
from jax import lax
from jax.experimental import pallas as pl
from jax.experimental.pallas import tpu as pltpu

N_DEV = 4
SQ = 1024
SKV = 1024
D_MODEL = 1024
H_PER = 8
H_HALF = 4
N_HEADS = 32
DH = 128
HALF = 512
NB = 4
R = 256
BLK = 64
SCALE = 0.08838834764831843


def _body(x_ref, wq_ref, k_ref, v_ref, wo_ref, out_ref,
          wqa, woa, wqb, wob, kbuf, vbuf, send_sems, recv_sems, kv_sems):
    my = lax.axis_index("i")
    left = lax.rem(my + N_DEV - 1, N_DEV)
    right = lax.rem(my + 1, N_DEV)

    wqa[0] = wq_ref[:, :HALF]
    wqb[0] = wq_ref[:, HALF:]
    woa[0] = wo_ref[:HALF, :]
    wob[0] = wo_ref[HALF:, :]

    barrier_sem = pltpu.get_barrier_semaphore()
    for nbr in (left, right):
        pl.semaphore_signal(
            barrier_sem, inc=1,
            device_id=(nbr,), device_id_type=pl.DeviceIdType.MESH,
        )
    pl.semaphore_wait(barrier_sem, 2)

    def head_index(s, h):
        if h < H_HALF:
            return lax.rem(my - s + N_DEV, N_DEV) * H_PER + h
        return lax.rem(my + s, N_DEV) * H_PER + h

    def kv_dma(t, slot):
        s, h = divmod(t, H_PER)
        head = head_index(s, h)
        copies = []
        for r in range(NB):
            for b in range(NB):
                src_lo = (NB * b + r) * BLK
                dst_lo = r * R + b * BLK
                ck = pltpu.make_async_copy(
                    k_ref.at[0, pl.ds(src_lo, BLK), head, :],
                    kbuf.at[slot, pl.ds(dst_lo, BLK), :],
                    kv_sems.at[slot, 0])
                cv = pltpu.make_async_copy(
                    v_ref.at[0, pl.ds(src_lo, BLK), head, :],
                    vbuf.at[slot, pl.ds(dst_lo, BLK), :],
                    kv_sems.at[slot, 1])
                ck.start()
                cv.start()
                copies.append(ck)
                copies.append(cv)
        return copies

    x = x_ref[0]
    pending = kv_dma(0, 0)
    hops = []
    for s in range(N_DEV):
        if s < N_DEV - 1:
            stage_hops = []
            for j, (buf, dev) in enumerate(
                    [(wqa, right), (woa, right), (wqb, left), (wob, left)]):
                rdma = pltpu.make_async_remote_copy(
                    src_ref=buf.at[s],
                    dst_ref=buf.at[s + 1],
                    send_sem=send_sems.at[s, j],
                    recv_sem=recv_sems.at[s, j],
                    device_id=(dev,),
                    device_id_type=pl.DeviceIdType.MESH,
                )
                rdma.start()
                stage_hops.append(rdma)
            hops.append(stage_hops)

        q_a = jnp.dot(x, wqa[s], preferred_element_type=jnp.float32)
        q_a = q_a.astype(jnp.bfloat16)
        q_b = jnp.dot(x, wqb[s], preferred_element_type=jnp.float32)
        q_b = q_b.astype(jnp.bfloat16)
        for h in range(H_PER):
            t = s * H_PER + h
            slot = t % 2
            for cpy in pending:
                cpy.wait()
            if t + 1 < N_HEADS:
                pending = kv_dma(t + 1, (t + 1) % 2)
            k_h = kbuf[slot].astype(jnp.bfloat16)
            v_h = vbuf[slot].astype(jnp.bfloat16)
            if h < H_HALF:
                q_h = q_a[:, h * DH:(h + 1) * DH]
                wo_h = woa[s, h * DH:(h + 1) * DH, :]
            else:
                q_h = q_b[:, (h - H_HALF) * DH:(h - H_HALF + 1) * DH]
                wo_h = wob[s, (h - H_HALF) * DH:(h - H_HALF + 1) * DH, :]
            for r in range(NB):
                q_r = q_h[r * R:(r + 1) * R, :]
                k_r = k_h[r * R:(r + 1) * R, :]
                v_r = v_h[r * R:(r + 1) * R, :]
                sc = jnp.dot(q_r, k_r.T, preferred_element_type=jnp.float32)
                sc = sc * SCALE
                m = jnp.max(sc, axis=-1, keepdims=True)
                w = jnp.exp(sc - m)
                w = (w / jnp.sum(w, axis=-1, keepdims=True)).astype(jnp.bfloat16)
                ctx = jnp.dot(w, v_r, preferred_element_type=jnp.float32)
                part = jnp.dot(ctx.astype(jnp.bfloat16), wo_h,
                               preferred_element_type=jnp.float32)
                if s == 0 and h == 0:
                    out_ref[0, r * R:(r + 1) * R, :] = part
                else:
                    out_ref[0, r * R:(r + 1) * R, :] += part

        if s < N_DEV - 1:
            for rdma in hops[s]:
                rdma.wait_recv()

    for stage_hops in hops:
        for rdma in stage_hops:
            rdma.wait_send()


def _permute_rows(a, axis):
    shape = a.shape
    pre, post = shape[:axis], shape[axis + 1:]
    a4 = a.reshape(*pre, NB, NB, BLK, *post)
    a4 = jnp.swapaxes(a4, axis, axis + 1)
    return a4.reshape(*shape)


def _unpermute_rows(a, axis):
    shape = a.shape
    pre, post = shape[:axis], shape[axis + 1:]
    a4 = a.reshape(*pre, NB, NB, BLK, *post)
    a4 = jnp.swapaxes(a4, axis, axis + 1)
    return a4.reshape(*shape)


def kernel(x, Wq, K_ext, V_ext, Wo):
    x16 = _permute_rows(x.astype(jnp.bfloat16), 1)
    wq16 = Wq.astype(jnp.bfloat16)
    wo16 = Wo.astype(jnp.bfloat16)
    out_p = pl.pallas_call(
        _body,
        out_shape=jax.ShapeDtypeStruct((1, SQ, D_MODEL), jnp.float32),
        in_specs=[
            pl.BlockSpec(memory_space=pltpu.VMEM),
            pl.BlockSpec(memory_space=pltpu.VMEM),
            pl.BlockSpec(memory_space=pl.ANY),
            pl.BlockSpec(memory_space=pl.ANY),
            pl.BlockSpec(memory_space=pltpu.VMEM),
        ],
        out_specs=pl.BlockSpec(memory_space=pltpu.VMEM),
        scratch_shapes=[
            pltpu.VMEM((N_DEV, D_MODEL, HALF), jnp.bfloat16),
            pltpu.VMEM((N_DEV, HALF, D_MODEL), jnp.bfloat16),
            pltpu.VMEM((N_DEV, D_MODEL, HALF), jnp.bfloat16),
            pltpu.VMEM((N_DEV, HALF, D_MODEL), jnp.bfloat16),
            pltpu.VMEM((2, SKV, DH), jnp.float32),
            pltpu.VMEM((2, SKV, DH), jnp.float32),
            pltpu.SemaphoreType.DMA((N_DEV - 1, 4)),
            pltpu.SemaphoreType.DMA((N_DEV - 1, 4)),
            pltpu.SemaphoreType.DMA((2, 2)),
        ],
        compiler_params=pltpu.CompilerParams(
            collective_id=0,
            vmem_limit_bytes=63 * 1024 * 1024,
        ),
    )(x16, wq16, K_ext, V_ext, wo16)
    return _unpermute_rows(out_p, 1)


# device time: 109972 ns/iter; 4.0262x vs baseline; 1.1663x over previous
import jax
import jax.numpy as jnp
from jax import lax
from jax.experimental import pallas as pl
from jax.experimental.pallas import tpu as pltpu

N_DEV = 4
SQ = 1024
SKV = 1024
D_MODEL = 1024
H_PER = 8
H_HALF = 4
N_HEADS = 32
DH = 128
HALF = 512
NB = 4
R = 256
BLK = 64
SCALE = 0.08838834764831843


def _body(x_ref, wq_ref, k_ref, v_ref, wo_ref, out_ref,
          wqa, woa, wqb, wob, kbuf, vbuf, send_sems, recv_sems, kv_sems):
    my = lax.axis_index("i")
    left = lax.rem(my + N_DEV - 1, N_DEV)
    right = lax.rem(my + 1, N_DEV)

    wqa[0] = wq_ref[:, :HALF]
    wqb[0] = wq_ref[:, HALF:]
    woa[0] = wo_ref[:HALF, :]
    wob[0] = wo_ref[HALF:, :]

    barrier_sem = pltpu.get_barrier_semaphore()
    for nbr in (left, right):
        pl.semaphore_signal(
            barrier_sem, inc=1,
            device_id=(nbr,), device_id_type=pl.DeviceIdType.MESH,
        )
    pl.semaphore_wait(barrier_sem, 2)

    def head_index(s, h):
        if h < H_HALF:
            return lax.rem(my - s + N_DEV, N_DEV) * H_PER + h
        return lax.rem(my + s, N_DEV) * H_PER + h

    def kv_dma(t, slot):
        s, h = divmod(t, H_PER)
        head = head_index(s, h)
        copies = []
        for r in range(NB):
            for b in range(NB):
                src_lo = (NB * b + r) * BLK
                dst_lo = r * R + b * BLK
                ck = pltpu.make_async_copy(
                    k_ref.at[0, pl.ds(src_lo, BLK), head, :],
                    kbuf.at[slot, pl.ds(dst_lo, BLK), :],
                    kv_sems.at[slot, 0])
                cv = pltpu.make_async_copy(
                    v_ref.at[0, pl.ds(src_lo, BLK), head, :],
                    vbuf.at[slot, pl.ds(dst_lo, BLK), :],
                    kv_sems.at[slot, 1])
                ck.start()
                cv.start()
                copies.append(ck)
                copies.append(cv)
        return copies

    x = x_ref[0]
    pending = kv_dma(0, 0)
    hops = []
    for s in range(N_DEV):
        if s < N_DEV - 1:
            stage_hops = []
            for j, (buf, dev) in enumerate(
                    [(wqa, right), (woa, right), (wqb, left), (wob, left)]):
                rdma = pltpu.make_async_remote_copy(
                    src_ref=buf.at[s],
                    dst_ref=buf.at[s + 1],
                    send_sem=send_sems.at[s, j],
                    recv_sem=recv_sems.at[s, j],
                    device_id=(dev,),
                    device_id_type=pl.DeviceIdType.MESH,
                )
                rdma.start()
                stage_hops.append(rdma)
            hops.append(stage_hops)

        q_a = jnp.dot(x, wqa[s], preferred_element_type=jnp.float32)
        q_a = q_a.astype(jnp.bfloat16)
        q_b = jnp.dot(x, wqb[s], preferred_element_type=jnp.float32)
        q_b = q_b.astype(jnp.bfloat16)
        ctx_a = [[] for _ in range(NB)]
        ctx_b = [[] for _ in range(NB)]
        for h in range(H_PER):
            t = s * H_PER + h
            slot = t % 2
            for cpy in pending:
                cpy.wait()
            if t + 1 < N_HEADS:
                pending = kv_dma(t + 1, (t + 1) % 2)
            k_h = kbuf[slot].astype(jnp.bfloat16)
            v_h = vbuf[slot].astype(jnp.bfloat16)
            if h < H_HALF:
                q_h = q_a[:, h * DH:(h + 1) * DH]
            else:
                q_h = q_b[:, (h - H_HALF) * DH:(h - H_HALF + 1) * DH]
            for r in range(NB):
                q_r = q_h[r * R:(r + 1) * R, :]
                k_r = k_h[r * R:(r + 1) * R, :]
                v_r = v_h[r * R:(r + 1) * R, :]
                sc = jnp.dot(q_r, k_r.T, preferred_element_type=jnp.float32)
                sc = sc * SCALE
                m = jnp.max(sc, axis=-1, keepdims=True)
                w = jnp.exp(sc - m)
                w = (w / jnp.sum(w, axis=-1, keepdims=True)).astype(jnp.bfloat16)
                ctx = jnp.dot(w, v_r, preferred_element_type=jnp.float32)
                (ctx_a if h < H_HALF else ctx_b)[r].append(
                    ctx.astype(jnp.bfloat16))
        for r in range(NB):
            ca = jnp.concatenate(ctx_a[r], axis=1)
            cb = jnp.concatenate(ctx_b[r], axis=1)
            part = (jnp.dot(ca, woa[s], preferred_element_type=jnp.float32)
                    + jnp.dot(cb, wob[s], preferred_element_type=jnp.float32))
            for b in range(NB):
                rows = pl.ds((NB * b + r) * BLK, BLK)
                if s == 0:
                    out_ref[0, rows, :] = part[b * BLK:(b + 1) * BLK, :]
                else:
                    out_ref[0, rows, :] += part[b * BLK:(b + 1) * BLK, :]

        if s < N_DEV - 1:
            for rdma in hops[s]:
                rdma.wait_recv()

    for stage_hops in hops:
        for rdma in stage_hops:
            rdma.wait_send()


def _permute_rows(a, axis):
    shape = a.shape
    pre, post = shape[:axis], shape[axis + 1:]
    a4 = a.reshape(*pre, NB, NB, BLK, *post)
    a4 = jnp.swapaxes(a4, axis, axis + 1)
    return a4.reshape(*shape)


def _unpermute_rows(a, axis):
    shape = a.shape
    pre, post = shape[:axis], shape[axis + 1:]
    a4 = a.reshape(*pre, NB, NB, BLK, *post)
    a4 = jnp.swapaxes(a4, axis, axis + 1)
    return a4.reshape(*shape)


def kernel(x, Wq, K_ext, V_ext, Wo):
    x16 = _permute_rows(x.astype(jnp.bfloat16), 1)
    wq16 = Wq.astype(jnp.bfloat16)
    wo16 = Wo.astype(jnp.bfloat16)
    out_p = pl.pallas_call(
        _body,
        out_shape=jax.ShapeDtypeStruct((1, SQ, D_MODEL), jnp.float32),
        in_specs=[
            pl.BlockSpec(memory_space=pltpu.VMEM),
            pl.BlockSpec(memory_space=pltpu.VMEM),
            pl.BlockSpec(memory_space=pl.ANY),
            pl.BlockSpec(memory_space=pl.ANY),
            pl.BlockSpec(memory_space=pltpu.VMEM),
        ],
        out_specs=pl.BlockSpec(memory_space=pltpu.VMEM),
        scratch_shapes=[
            pltpu.VMEM((N_DEV, D_MODEL, HALF), jnp.bfloat16),
            pltpu.VMEM((N_DEV, HALF, D_MODEL), jnp.bfloat16),
            pltpu.VMEM((N_DEV, D_MODEL, HALF), jnp.bfloat16),
            pltpu.VMEM((N_DEV, HALF, D_MODEL), jnp.bfloat16),
            pltpu.VMEM((2, SKV, DH), jnp.float32),
            pltpu.VMEM((2, SKV, DH), jnp.float32),
            pltpu.SemaphoreType.DMA((N_DEV - 1, 4)),
            pltpu.SemaphoreType.DMA((N_DEV - 1, 4)),
            pltpu.SemaphoreType.DMA((2, 2)),
        ],
        compiler_params=pltpu.CompilerParams(
            collective_id=0,
            vmem_limit_bytes=63 * 1024 * 1024,
        ),
    )(x16, wq16, K_ext, V_ext, wo16)
    return out_p


# device time: 107941 ns/iter; 4.1019x vs baseline; 1.0188x over previous
import jax
import jax.numpy as jnp
from jax import lax
from jax.experimental import pallas as pl
from jax.experimental.pallas import tpu as pltpu

N_DEV = 4
SQ = 1024
SKV = 1024
D_MODEL = 1024
H_PER = 8
H_HALF = 4
N_HEADS = 32
DH = 128
HALF = 512
NB = 4
R = 256
BLK = 64
SCALE = 0.08838834764831843


def _body(x_ref, wq_ref, k_ref, v_ref, wo_ref, out_ref,
          wqa, woa, wqb, wob, kbuf, vbuf, send_sems, recv_sems, kv_sems):
    my = lax.axis_index("i")
    left = lax.rem(my + N_DEV - 1, N_DEV)
    right = lax.rem(my + 1, N_DEV)

    wqa[0] = wq_ref[:, :HALF]
    wqb[0] = wq_ref[:, HALF:]
    woa[0] = wo_ref[:HALF, :]
    wob[0] = wo_ref[HALF:, :]

    barrier_sem = pltpu.get_barrier_semaphore()
    for nbr in (left, right):
        pl.semaphore_signal(
            barrier_sem, inc=1,
            device_id=(nbr,), device_id_type=pl.DeviceIdType.MESH,
        )
    pl.semaphore_wait(barrier_sem, 2)

    def head_index(s, h):
        if h < H_HALF:
            return lax.rem(my - s + N_DEV, N_DEV) * H_PER + h
        return lax.rem(my + s, N_DEV) * H_PER + h

    def kv_dma(t, slot):
        s, h = divmod(t, H_PER)
        head = head_index(s, h)
        copies = []
        for r in range(NB):
            for b in range(NB):
                src_lo = (NB * b + r) * BLK
                dst_lo = r * R + b * BLK
                ck = pltpu.make_async_copy(
                    k_ref.at[0, pl.ds(src_lo, BLK), head, :],
                    kbuf.at[slot, pl.ds(dst_lo, BLK), :],
                    kv_sems.at[slot, 0])
                cv = pltpu.make_async_copy(
                    v_ref.at[0, pl.ds(src_lo, BLK), head, :],
                    vbuf.at[slot, pl.ds(dst_lo, BLK), :],
                    kv_sems.at[slot, 1])
                ck.start()
                cv.start()
                copies.append(ck)
                copies.append(cv)
        return copies

    x = x_ref[0]
    pending = {}
    for t0 in range(3):
        pending[t0] = kv_dma(t0, t0 % 4)
    hops = []
    for s in range(N_DEV):
        if s < N_DEV - 1:
            stage_hops = []
            for j, (buf, dev) in enumerate(
                    [(wqa, right), (woa, right), (wqb, left), (wob, left)]):
                rdma = pltpu.make_async_remote_copy(
                    src_ref=buf.at[s],
                    dst_ref=buf.at[s + 1],
                    send_sem=send_sems.at[s, j],
                    recv_sem=recv_sems.at[s, j],
                    device_id=(dev,),
                    device_id_type=pl.DeviceIdType.MESH,
                )
                rdma.start()
                stage_hops.append(rdma)
            hops.append(stage_hops)

        q_a = jnp.dot(x, wqa[s], preferred_element_type=jnp.float32)
        q_a = q_a.astype(jnp.bfloat16)
        q_b = jnp.dot(x, wqb[s], preferred_element_type=jnp.float32)
        q_b = q_b.astype(jnp.bfloat16)
        ctx_a = [[] for _ in range(NB)]
        ctx_b = [[] for _ in range(NB)]
        for h in range(H_PER):
            t = s * H_PER + h
            slot = t % 4
            for cpy in pending.pop(t):
                cpy.wait()
            if t + 3 < N_HEADS:
                pending[t + 3] = kv_dma(t + 3, (t + 3) % 4)
            k_h = kbuf[slot].astype(jnp.bfloat16)
            v_h = vbuf[slot].astype(jnp.bfloat16)
            if h < H_HALF:
                q_h = q_a[:, h * DH:(h + 1) * DH]
            else:
                q_h = q_b[:, (h - H_HALF) * DH:(h - H_HALF + 1) * DH]
            for r in range(NB):
                q_r = q_h[r * R:(r + 1) * R, :]
                k_r = k_h[r * R:(r + 1) * R, :]
                v_r = v_h[r * R:(r + 1) * R, :]
                sc = jnp.dot(q_r, k_r.T, preferred_element_type=jnp.float32)
                w = jnp.exp(sc * SCALE)
                w = (w / jnp.sum(w, axis=-1, keepdims=True)).astype(jnp.bfloat16)
                ctx = jnp.dot(w, v_r, preferred_element_type=jnp.float32)
                (ctx_a if h < H_HALF else ctx_b)[r].append(
                    ctx.astype(jnp.bfloat16))
        for r in range(NB):
            ca = jnp.concatenate(ctx_a[r], axis=1)
            cb = jnp.concatenate(ctx_b[r], axis=1)
            part = (jnp.dot(ca, woa[s], preferred_element_type=jnp.float32)
                    + jnp.dot(cb, wob[s], preferred_element_type=jnp.float32))
            for b in range(NB):
                rows = pl.ds((NB * b + r) * BLK, BLK)
                if s == 0:
                    out_ref[0, rows, :] = part[b * BLK:(b + 1) * BLK, :]
                else:
                    out_ref[0, rows, :] += part[b * BLK:(b + 1) * BLK, :]

        if s < N_DEV - 1:
            for rdma in hops[s]:
                rdma.wait_recv()

    for stage_hops in hops:
        for rdma in stage_hops:
            rdma.wait_send()


def _permute_rows(a, axis):
    shape = a.shape
    pre, post = shape[:axis], shape[axis + 1:]
    a4 = a.reshape(*pre, NB, NB, BLK, *post)
    a4 = jnp.swapaxes(a4, axis, axis + 1)
    return a4.reshape(*shape)


def _unpermute_rows(a, axis):
    shape = a.shape
    pre, post = shape[:axis], shape[axis + 1:]
    a4 = a.reshape(*pre, NB, NB, BLK, *post)
    a4 = jnp.swapaxes(a4, axis, axis + 1)
    return a4.reshape(*shape)


def kernel(x, Wq, K_ext, V_ext, Wo):
    x16 = _permute_rows(x.astype(jnp.bfloat16), 1)
    wq16 = Wq.astype(jnp.bfloat16)
    wo16 = Wo.astype(jnp.bfloat16)
    out_p = pl.pallas_call(
        _body,
        out_shape=jax.ShapeDtypeStruct((1, SQ, D_MODEL), jnp.float32),
        in_specs=[
            pl.BlockSpec(memory_space=pltpu.VMEM),
            pl.BlockSpec(memory_space=pltpu.VMEM),
            pl.BlockSpec(memory_space=pl.ANY),
            pl.BlockSpec(memory_space=pl.ANY),
            pl.BlockSpec(memory_space=pltpu.VMEM),
        ],
        out_specs=pl.BlockSpec(memory_space=pltpu.VMEM),
        scratch_shapes=[
            pltpu.VMEM((N_DEV, D_MODEL, HALF), jnp.bfloat16),
            pltpu.VMEM((N_DEV, HALF, D_MODEL), jnp.bfloat16),
            pltpu.VMEM((N_DEV, D_MODEL, HALF), jnp.bfloat16),
            pltpu.VMEM((N_DEV, HALF, D_MODEL), jnp.bfloat16),
            pltpu.VMEM((4, SKV, DH), jnp.float32),
            pltpu.VMEM((4, SKV, DH), jnp.float32),
            pltpu.SemaphoreType.DMA((N_DEV - 1, 4)),
            pltpu.SemaphoreType.DMA((N_DEV - 1, 4)),
            pltpu.SemaphoreType.DMA((4, 2)),
        ],
        compiler_params=pltpu.CompilerParams(
            collective_id=0,
            vmem_limit_bytes=63 * 1024 * 1024,
        ),
    )(x16, wq16, K_ext, V_ext, wo16)
    return out_p


# device time: 107391 ns/iter; 4.1230x vs baseline; 1.0051x over previous
import jax
import jax.numpy as jnp
from jax import lax
from jax.experimental import pallas as pl
from jax.experimental.pallas import tpu as pltpu

N_DEV = 4
SQ = 1024
SKV = 1024
D_MODEL = 1024
H_PER = 8
H_HALF = 4
N_HEADS = 32
DH = 128
HALF = 512
SUB = 4
NB = 4
R = 256
BLK = 64
SCALE = 0.08838834764831843


def _body(x_ref, wq_ref, k_ref, v_ref, wo_ref, out_ref,
          wqa, woa, wqb, wob, kbuf, vbuf, send_sems, recv_sems, kv_sems):
    my = lax.axis_index("i")
    left = lax.rem(my + N_DEV - 1, N_DEV)
    right = lax.rem(my + 1, N_DEV)

    wqa[0] = wq_ref[:, :HALF]
    wqb[0] = wq_ref[:, HALF:]
    woa[0] = wo_ref[:HALF, :]
    wob[0] = wo_ref[HALF:, :]

    barrier_sem = pltpu.get_barrier_semaphore()
    for nbr in (left, right):
        pl.semaphore_signal(
            barrier_sem, inc=1,
            device_id=(nbr,), device_id_type=pl.DeviceIdType.MESH,
        )
    pl.semaphore_wait(barrier_sem, 2)

    def head_index(s, h):
        if h < H_HALF:
            return lax.rem(my - s + N_DEV, N_DEV) * H_PER + h
        return lax.rem(my + s, N_DEV) * H_PER + h

    def kv_dma(t, slot):
        s, h = divmod(t, H_PER)
        head = head_index(s, h)
        copies = []
        for r in range(NB):
            for b in range(NB):
                src_lo = (NB * b + r) * BLK
                dst_lo = r * R + b * BLK
                ck = pltpu.make_async_copy(
                    k_ref.at[0, pl.ds(src_lo, BLK), head, :],
                    kbuf.at[slot, pl.ds(dst_lo, BLK), :],
                    kv_sems.at[slot, 0])
                cv = pltpu.make_async_copy(
                    v_ref.at[0, pl.ds(src_lo, BLK), head, :],
                    vbuf.at[slot, pl.ds(dst_lo, BLK), :],
                    kv_sems.at[slot, 1])
                ck.start()
                cv.start()
                copies.append(ck)
                copies.append(cv)
        return copies

    x = x_ref[0]
    pending = {}
    for t0 in range(3):
        pending[t0] = kv_dma(t0, t0 % 4)

    streams = [(wqa, D_MODEL), (woa, HALF), (wqb, D_MODEL), (wob, HALF)]

    def make_T(s, j):
        rdmas = []
        for st, (buf, rows) in enumerate(streams):
            sub = rows // SUB
            dev = right if st < 2 else left
            rdmas.append(pltpu.make_async_remote_copy(
                src_ref=buf.at[s, pl.ds(j * sub, sub), :],
                dst_ref=buf.at[s + 1, pl.ds(j * sub, sub), :],
                send_sem=send_sems.at[s, st, j],
                recv_sem=recv_sems.at[s, st, j],
                device_id=(dev,),
                device_id_type=pl.DeviceIdType.MESH,
            ))
        return rdmas

    transfers = {}

    def start_T(s, j):
        transfers[(s, j)] = make_T(s, j)
        for rdma in transfers[(s, j)]:
            rdma.start()

    for s in range(N_DEV):
        if s == 0:
            for j in range(SUB):
                start_T(0, j)

        q_a = jnp.dot(x, wqa[s], preferred_element_type=jnp.float32)
        q_a = q_a.astype(jnp.bfloat16)
        q_b = jnp.dot(x, wqb[s], preferred_element_type=jnp.float32)
        q_b = q_b.astype(jnp.bfloat16)
        ctx_a = [[] for _ in range(NB)]
        ctx_b = [[] for _ in range(NB)]
        for h in range(H_PER):
            t = s * H_PER + h
            slot = t % 4
            for cpy in pending.pop(t):
                cpy.wait()
            if t + 3 < N_HEADS:
                pending[t + 3] = kv_dma(t + 3, (t + 3) % 4)
            k_h = kbuf[slot].astype(jnp.bfloat16)
            v_h = vbuf[slot].astype(jnp.bfloat16)
            if h < H_HALF:
                q_h = q_a[:, h * DH:(h + 1) * DH]
            else:
                q_h = q_b[:, (h - H_HALF) * DH:(h - H_HALF + 1) * DH]
            for r in range(NB):
                q_r = q_h[r * R:(r + 1) * R, :]
                k_r = k_h[r * R:(r + 1) * R, :]
                v_r = v_h[r * R:(r + 1) * R, :]
                sc = jnp.dot(q_r, k_r.T, preferred_element_type=jnp.float32)
                w = jnp.exp(sc * SCALE)
                w = (w / jnp.sum(w, axis=-1, keepdims=True)).astype(jnp.bfloat16)
                ctx = jnp.dot(w, v_r, preferred_element_type=jnp.float32)
                (ctx_a if h < H_HALF else ctx_b)[r].append(
                    ctx.astype(jnp.bfloat16))
            if s < N_DEV - 1 and h % 2 == 1:
                j = h // 2
                for rdma in transfers[(s, j)]:
                    rdma.wait_recv()
                if s < N_DEV - 2:
                    start_T(s + 1, j)
        for r in range(NB):
            ca = jnp.concatenate(ctx_a[r], axis=1)
            cb = jnp.concatenate(ctx_b[r], axis=1)
            part = (jnp.dot(ca, woa[s], preferred_element_type=jnp.float32)
                    + jnp.dot(cb, wob[s], preferred_element_type=jnp.float32))
            for b in range(NB):
                rows = pl.ds((NB * b + r) * BLK, BLK)
                if s == 0:
                    out_ref[0, rows, :] = part[b * BLK:(b + 1) * BLK, :]
                else:
                    out_ref[0, rows, :] += part[b * BLK:(b + 1) * BLK, :]

    for rdmas in transfers.values():
        for rdma in rdmas:
            rdma.wait_send()


def _permute_rows(a, axis):
    shape = a.shape
    pre, post = shape[:axis], shape[axis + 1:]
    a4 = a.reshape(*pre, NB, NB, BLK, *post)
    a4 = jnp.swapaxes(a4, axis, axis + 1)
    return a4.reshape(*shape)


def _unpermute_rows(a, axis):
    shape = a.shape
    pre, post = shape[:axis], shape[axis + 1:]
    a4 = a.reshape(*pre, NB, NB, BLK, *post)
    a4 = jnp.swapaxes(a4, axis, axis + 1)
    return a4.reshape(*shape)


def kernel(x, Wq, K_ext, V_ext, Wo):
    x16 = _permute_rows(x.astype(jnp.bfloat16), 1)
    wq16 = Wq.astype(jnp.bfloat16)
    wo16 = Wo.astype(jnp.bfloat16)
    out_p = pl.pallas_call(
        _body,
        out_shape=jax.ShapeDtypeStruct((1, SQ, D_MODEL), jnp.float32),
        in_specs=[
            pl.BlockSpec(memory_space=pltpu.VMEM),
            pl.BlockSpec(memory_space=pltpu.VMEM),
            pl.BlockSpec(memory_space=pl.ANY),
            pl.BlockSpec(memory_space=pl.ANY),
            pl.BlockSpec(memory_space=pltpu.VMEM),
        ],
        out_specs=pl.BlockSpec(memory_space=pltpu.VMEM),
        scratch_shapes=[
            pltpu.VMEM((N_DEV, D_MODEL, HALF), jnp.bfloat16),
            pltpu.VMEM((N_DEV, HALF, D_MODEL), jnp.bfloat16),
            pltpu.VMEM((N_DEV, D_MODEL, HALF), jnp.bfloat16),
            pltpu.VMEM((N_DEV, HALF, D_MODEL), jnp.bfloat16),
            pltpu.VMEM((4, SKV, DH), jnp.float32),
            pltpu.VMEM((4, SKV, DH), jnp.float32),
            pltpu.SemaphoreType.DMA((N_DEV - 1, 4, SUB)),
            pltpu.SemaphoreType.DMA((N_DEV - 1, 4, SUB)),
            pltpu.SemaphoreType.DMA((4, 2)),
        ],
        compiler_params=pltpu.CompilerParams(
            collective_id=0,
            vmem_limit_bytes=63 * 1024 * 1024,
        ),
    )(x16, wq16, K_ext, V_ext, wo16)
    return out_p
